# baseline (speedup 1.0000x reference)
import jax
import jax.numpy as jnp
import numpy as np
from jax.experimental import pallas as pl
from jax.experimental.pallas import tpu as pltpu

_C, _H, _W = 3, 32, 32
_KH = _KW = 8
_HP, _WP = _H // _KH, _W // _KW
_P = _HP * _WP
_O = 32
_F = 32
_LOW = 7
_KFLAT = _C * _H * _W
_NH = _P * _O
_KSLAB = _KH * _W
_VROWS = _C * _KSLAB
_VCOLS = _WP * _O
_OUT_W = 2 * _F + _LOW

_IMAGENET_MEAN = jnp.array([0.485, 0.456, 0.406], jnp.float32)
_IMAGENET_STD = jnp.array([0.229, 0.224, 0.225], jnp.float32)


def _conv_block(x, v):
    hs = []
    for hp in range(_HP):
        acc = None
        for c in range(_C):
            lo = c * _H * _W + hp * _KSLAB
            d = jnp.dot(x[:, lo:lo + _KSLAB], v[c * _KSLAB:(c + 1) * _KSLAB],
                        preferred_element_type=jnp.float32)
            acc = d if acc is None else acc + d
        hs.append(acc)
    return jnp.concatenate(hs, axis=1)


def _enc_kernel(x0_ref, x1_ref, a_ref, v_ref, bf_ref, wl_ref, bl_ref,
                out_ref):
    v = v_ref[...]
    bb = jnp.tile(bf_ref[...], (1, _P))
    w2 = jnp.tile(wl_ref[...] * (1.0 / _P), (_P, 1)).astype(jnp.bfloat16)
    bl = bl_ref[...]

    h0 = _conv_block(x0_ref[...].astype(jnp.bfloat16), v)
    h0 = jnp.maximum(h0 + bb, 0.0).astype(jnp.bfloat16)
    h1 = _conv_block(x1_ref[...].astype(jnp.bfloat16), v)
    h1 = jnp.maximum(h1 + bb, 0.0).astype(jnp.bfloat16)
    f0 = jnp.dot(h0, w2, preferred_element_type=jnp.float32) + bl
    f1 = jnp.dot(h1, w2, preferred_element_type=jnp.float32) + bl
    out_ref[...] = jnp.concatenate([f0, f1, a_ref[...]], axis=1)


def _patch_col_mask():
    r = np.arange(_VROWS)
    wp_row = (r % _W) // _KW
    wp_col = np.arange(_VCOLS) // _O
    return (wp_row[:, None] == wp_col[None, :]).astype(np.float32)


_MASK = _patch_col_mask()


def _build_conv_matrix(w_conv, b_conv):
    w_fold = w_conv / _IMAGENET_STD[None, :, None, None]
    b_fold = b_conv - jnp.einsum(
        "ochw,c->o", w_conv, _IMAGENET_MEAN / _IMAGENET_STD)

    base = jnp.transpose(w_fold, (1, 2, 3, 0)).reshape(_C, _KH, 1, _KW, _O)
    e = jnp.broadcast_to(
        base, (_C, _KH, _WP, _KW, _O)).reshape(_VROWS, _O)
    v = (jnp.tile(e, (1, _WP)) * _MASK).astype(jnp.bfloat16)
    return v, b_fold.reshape(1, _O)


def kernel(camera_0, camera_1, agent_pos, w_conv, b_conv, w_lin, b_lin):
    batch = camera_0.shape[0]
    x0 = camera_0.reshape(batch, _KFLAT)
    x1 = camera_1.reshape(batch, _KFLAT)
    a = agent_pos.astype(jnp.float32)

    v, bf = _build_conv_matrix(w_conv, b_conv)
    wl = w_lin
    bl = b_lin.reshape(1, _F)

    vm = pltpu.VMEM
    out = pl.pallas_call(
        _enc_kernel,
        out_shape=jax.ShapeDtypeStruct((batch, _OUT_W), jnp.float32),
        in_specs=[
            pl.BlockSpec(memory_space=vm),
            pl.BlockSpec(memory_space=vm),
            pl.BlockSpec(memory_space=vm),
            pl.BlockSpec(memory_space=vm),
            pl.BlockSpec(memory_space=vm),
            pl.BlockSpec(memory_space=vm),
            pl.BlockSpec(memory_space=vm),
        ],
        out_specs=pl.BlockSpec(memory_space=vm),
        compiler_params=pltpu.CompilerParams(
            vmem_limit_bytes=56 * 1024 * 1024,
        ),
    )(x0, x1, a, v, bf, wl, bl)

    return out


# speedup vs baseline: 1.0618x; 1.0618x over previous
import jax
import jax.numpy as jnp
import numpy as np
from jax.experimental import pallas as pl
from jax.experimental.pallas import tpu as pltpu

_C, _H, _W = 3, 32, 32
_KH = _KW = 8
_HP, _WP = _H // _KH, _W // _KW
_P = _HP * _WP
_O = 32
_F = 32
_LOW = 7
_KFLAT = _C * _H * _W
_NH = _P * _O
_KSLAB = _KH * _W
_VROWS = _C * _KSLAB
_VCOLS = _WP * _O
_OUT_W = 2 * _F + _LOW
_BT = 256

_IMAGENET_MEAN = jnp.array([0.485, 0.456, 0.406], jnp.float32)
_IMAGENET_STD = jnp.array([0.229, 0.224, 0.225], jnp.float32)


def _conv_block(x, v):
    hs = []
    for hp in range(_HP):
        acc = None
        for c in range(_C):
            lo = c * _H * _W + hp * _KSLAB
            d = jnp.dot(x[:, lo:lo + _KSLAB], v[c * _KSLAB:(c + 1) * _KSLAB],
                        preferred_element_type=jnp.float32)
            acc = d if acc is None else acc + d
        hs.append(acc)
    return jnp.concatenate(hs, axis=1)


def _enc_kernel(x0_ref, x1_ref, a_ref, v_ref, bf_ref, wl_ref, bl_ref,
                out_ref):
    v = v_ref[...]
    bb = jnp.tile(bf_ref[...], (1, _P))
    w2 = jnp.tile(wl_ref[...] * (1.0 / _P), (_P, 1)).astype(jnp.bfloat16)
    bl = bl_ref[...]

    h0 = _conv_block(x0_ref[...].astype(jnp.bfloat16), v)
    h0 = jnp.maximum(h0 + bb, 0.0).astype(jnp.bfloat16)
    h1 = _conv_block(x1_ref[...].astype(jnp.bfloat16), v)
    h1 = jnp.maximum(h1 + bb, 0.0).astype(jnp.bfloat16)
    f0 = jnp.dot(h0, w2, preferred_element_type=jnp.float32) + bl
    f1 = jnp.dot(h1, w2, preferred_element_type=jnp.float32) + bl
    out_ref[...] = jnp.concatenate([f0, f1, a_ref[...]], axis=1)


def _patch_col_mask():
    r = np.arange(_VROWS)
    wp_row = (r % _W) // _KW
    wp_col = np.arange(_VCOLS) // _O
    return (wp_row[:, None] == wp_col[None, :]).astype(np.float32)


_MASK = _patch_col_mask()


def _build_conv_matrix(w_conv, b_conv):
    w_fold = w_conv / _IMAGENET_STD[None, :, None, None]
    b_fold = b_conv - jnp.einsum(
        "ochw,c->o", w_conv, _IMAGENET_MEAN / _IMAGENET_STD)

    base = jnp.transpose(w_fold, (1, 2, 3, 0)).reshape(_C, _KH, 1, _KW, _O)
    e = jnp.broadcast_to(
        base, (_C, _KH, _WP, _KW, _O)).reshape(_VROWS, _O)
    v = (jnp.tile(e, (1, _WP)) * _MASK).astype(jnp.bfloat16)
    return v, b_fold.reshape(1, _O)


def kernel(camera_0, camera_1, agent_pos, w_conv, b_conv, w_lin, b_lin):
    batch = camera_0.shape[0]
    x0 = camera_0.reshape(batch, _KFLAT)
    x1 = camera_1.reshape(batch, _KFLAT)
    a = agent_pos.astype(jnp.float32)

    v, bf = _build_conv_matrix(w_conv, b_conv)
    wl = w_lin
    bl = b_lin.reshape(1, _F)

    bt = min(_BT, pl.cdiv(batch, 8) * 8)
    n_pad = pl.cdiv(batch, bt) * bt
    if n_pad != batch:
        x0 = jnp.pad(x0, ((0, n_pad - batch), (0, 0)))
        x1 = jnp.pad(x1, ((0, n_pad - batch), (0, 0)))
        a = jnp.pad(a, ((0, n_pad - batch), (0, 0)))

    out = pl.pallas_call(
        _enc_kernel,
        out_shape=jax.ShapeDtypeStruct((n_pad, _OUT_W), jnp.float32),
        grid=(n_pad // bt,),
        in_specs=[
            pl.BlockSpec((bt, _KFLAT), lambda i: (i, 0)),
            pl.BlockSpec((bt, _KFLAT), lambda i: (i, 0)),
            pl.BlockSpec((bt, _LOW), lambda i: (i, 0)),
            pl.BlockSpec((_VROWS, _VCOLS), lambda i: (0, 0)),
            pl.BlockSpec((1, _O), lambda i: (0, 0)),
            pl.BlockSpec((_O, _F), lambda i: (0, 0)),
            pl.BlockSpec((1, _F), lambda i: (0, 0)),
        ],
        out_specs=pl.BlockSpec((bt, _OUT_W), lambda i: (i, 0)),
        compiler_params=pltpu.CompilerParams(
            dimension_semantics=("parallel",),
        ),
    )(x0, x1, a, v, bf, wl, bl)

    return out[:batch]
